# Initial kernel scaffold; baseline (speedup 1.0000x reference)
import jax
import jax.numpy as jnp
from jax.experimental import pallas as pl


def kernel(x0, x1, edges1, edges2, pos0, pos1, Wq, Wk, Wv, Wo, W1, b1, W2, b2):
    raise NotImplementedError("write your pallas kernel here")


# speedup vs baseline: 21.9489x; 21.9489x over previous
import functools

import jax
import jax.numpy as jnp
import numpy as np
from jax import lax
from jax.experimental import pallas as pl
from jax.experimental.pallas import tpu as pltpu
from jax.experimental.pallas import tpu_sc as plsc

N = 49152
E = 589824
D = 128
H = 4
DH = 32

NC, NS = 2, 16
NW = NC * NS
C = 128
EPW = E // NW
GCHUNKS = EPW // C
EPT = E // NS
SCHUNKS = EPT // C
AW = 128
CH = C // 2
NWIN = 4
NH = N // NWIN
ACC_ROWS = NH + 1024
ZPT = ACC_ROWS // NS
DPT = NH // NS

NB1 = 2048
EB = 4096
NB5 = 2048
PB = NB5 // 4

_f32 = jnp.float32


def _qkv_body(x_ref, wq_ref, wk_ref, wv_ref, q_ref, kv_ref):
    x = x_ref[...]
    q_ref[...] = jnp.dot(x, wq_ref[...], preferred_element_type=_f32)
    k = jnp.dot(x, wk_ref[...], preferred_element_type=_f32)
    v = jnp.dot(x, wv_ref[...], preferred_element_type=_f32)
    kv_ref[...] = jnp.concatenate([k, v], axis=-1)


def _qkv(xT, Wq, Wk, Wv):
    grid = (N // NB1,)
    wspec = pl.BlockSpec((D, D), lambda i: (0, 0))
    return pl.pallas_call(
        _qkv_body,
        grid=grid,
        in_specs=[pl.BlockSpec((NB1, D), lambda i: (i, 0)), wspec, wspec, wspec],
        out_specs=[pl.BlockSpec((NB1, D), lambda i: (i, 0)),
                   pl.BlockSpec((NB1, 2 * D), lambda i: (i, 0))],
        out_shape=[jax.ShapeDtypeStruct((N, D), _f32),
                   jax.ShapeDtypeStruct((N, 2 * D), _f32)],
    )(xT, Wq, Wk, Wv)


def _gather_body(q_hbm, kv_hbm, dsti_hbm, srci_hbm, qd_hbm, kvs_hbm,
                 didx, sidx, qbuf, kvbuf, sem_q, sem_kv):
    wid = lax.axis_index("s") * NC + lax.axis_index("c")
    pltpu.sync_copy(dsti_hbm.at[wid], didx)
    pltpu.sync_copy(srci_hbm.at[wid], sidx)

    def step(j, carry):
        base = wid * EPW + j * C
        cq = pltpu.async_copy(q_hbm.at[didx.at[j]], qbuf, sem_q)
        ckv = pltpu.async_copy(kv_hbm.at[sidx.at[j]], kvbuf, sem_kv)
        cq.wait()
        pltpu.sync_copy(qbuf, qd_hbm.at[pl.ds(base, C)])
        ckv.wait()
        pltpu.sync_copy(kvbuf, kvs_hbm.at[pl.ds(base, C)])
        return carry

    lax.fori_loop(0, GCHUNKS, step, 0)


def _gather(q, kv, dsti, srci):
    mesh = plsc.VectorSubcoreMesh(core_axis_name="c", subcore_axis_name="s",
                                  num_cores=NC, num_subcores=NS)
    f = pl.kernel(
        _gather_body,
        out_type=[jax.ShapeDtypeStruct((E, D), _f32),
                  jax.ShapeDtypeStruct((E, 2 * D), _f32)],
        mesh=mesh,
        scratch_types=[
            pltpu.VMEM((GCHUNKS, C), jnp.int32),
            pltpu.VMEM((GCHUNKS, C), jnp.int32),
            pltpu.VMEM((C, D), _f32),
            pltpu.VMEM((C, 2 * D), _f32),
            pltpu.SemaphoreType.DMA,
            pltpu.SemaphoreType.DMA,
        ],
    )
    return f(q, kv, dsti, srci)


def _score_body(qd_ref, kvs_ref, b_ref, r_ref, out_ref):
    qd = qd_ref[...]
    kvs = kvs_ref[...]
    k = kvs[:, :D]
    v = kvs[:, D:]
    prod = qd * k
    s = jnp.dot(prod, b_ref[...], preferred_element_type=_f32) * (1.0 / np.sqrt(DH))
    e = jnp.exp(s)
    eexp = jnp.dot(e, r_ref[...], preferred_element_type=_f32)
    wm = v * eexp
    zz = jnp.zeros((EB, DH - 1), _f32)
    for c in range(NC):
        row = jnp.concatenate(
            [wm[:, DH * 2 * c:DH * (2 * c + 1)], e[:, 2 * c:2 * c + 1], zz,
             wm[:, DH * (2 * c + 1):DH * (2 * c + 2)],
             e[:, 2 * c + 1:2 * c + 2], zz], axis=-1)
        out_ref[c] = row


def _score(qd, kvs, Bsel, Rsel):
    grid = (E // EB,)
    return pl.pallas_call(
        _score_body,
        grid=grid,
        in_specs=[pl.BlockSpec((EB, D), lambda i: (i, 0)),
                  pl.BlockSpec((EB, 2 * D), lambda i: (i, 0)),
                  pl.BlockSpec((D, H), lambda i: (0, 0)),
                  pl.BlockSpec((H, D), lambda i: (0, 0))],
        out_specs=pl.BlockSpec((NC, EB, AW), lambda i: (0, i, 0)),
        out_shape=jax.ShapeDtypeStruct((NC, E, AW), _f32),
    )(qd, kvs, Bsel, Rsel)


def _scatter_body(wma_hbm, dsti_hbm, z_hbm, out_hbm,
                  idxc, idxta, idxtb, buf, zv, acc):
    c = lax.axis_index("c")
    sid = lax.axis_index("s")
    pltpu.sync_copy(z_hbm, zv)
    for w in range(NWIN):
        base = w * NH

        def zero_step(i, carry):
            pltpu.sync_copy(zv, acc.at[pl.ds(sid * ZPT + i * CH, CH)])
            return carry

        lax.fori_loop(0, ZPT // CH, zero_step, 0)
        plsc.subcore_barrier()

        def step(j, carry):
            e0 = sid * EPT + j * C
            pltpu.sync_copy(dsti_hbm.at[sid, j], idxc)
            for u in range(CH // 16):
                a = idxc[pl.ds(u * 16, 16)]
                t = a - base
                ok = (t >= 0) & (t < NH)
                idxta[pl.ds(u * 16, 16)] = jnp.where(ok, t, NH)
                a2 = idxc[pl.ds(CH + u * 16, 16)]
                t2 = a2 - base
                ok2 = (t2 >= 0) & (t2 < NH)
                idxtb[pl.ds(u * 16, 16)] = jnp.where(ok2, t2, NH)
            pltpu.sync_copy(wma_hbm.at[c, pl.ds(e0, CH)], buf)
            pltpu.sync_copy(buf, acc.at[idxta], add=True)
            pltpu.sync_copy(wma_hbm.at[c, pl.ds(e0 + CH, CH)], buf)
            pltpu.sync_copy(buf, acc.at[idxtb], add=True)
            return carry

        lax.fori_loop(0, SCHUNKS, step, 0)
        plsc.subcore_barrier()
        pltpu.sync_copy(acc.at[pl.ds(sid * DPT, DPT)],
                        out_hbm.at[c, pl.ds(base + sid * DPT, DPT)])
        plsc.subcore_barrier()


def _scatter(wma, dsti, zrows):
    mesh = plsc.VectorSubcoreMesh(core_axis_name="c", subcore_axis_name="s",
                                  num_cores=NC, num_subcores=NS)
    f = pl.kernel(
        _scatter_body,
        out_type=jax.ShapeDtypeStruct((NC, N, AW), _f32),
        mesh=mesh,
        scratch_types=[
            pltpu.VMEM((C,), jnp.int32),
            pltpu.VMEM((CH,), jnp.int32),
            pltpu.VMEM((CH,), jnp.int32),
            pltpu.VMEM((CH, AW), _f32),
            pltpu.VMEM((CH, AW), _f32),
            pltpu.VMEM_SHARED((ACC_ROWS, AW), _f32),
        ],
    )
    return f(wma, dsti, zrows)


def _post_body(x_ref, msgu_ref, pos_ref, wo_ref, w1a_ref, w1b_ref, b1_ref,
               w2_ref, b2_ref, pool_ref, feat_ref, pos_out_ref):
    x = x_ref[...]
    mu = msgu_ref[...]
    pieces = []
    for c in range(NC):
        bh = mu[c]
        for p in range(2):
            num = bh[:, 64 * p:64 * p + DH]
            den = bh[:, 64 * p + DH:64 * p + DH + 1] + 1e-9
            pieces.append(num / den)
    msg = jnp.concatenate(pieces, axis=-1)
    mo = jnp.dot(msg, wo_ref[...], preferred_element_type=_f32)
    h1 = jnp.dot(x, w1a_ref[...], preferred_element_type=_f32)
    h1 = h1 + jnp.dot(mo, w1b_ref[...], preferred_element_type=_f32)
    h1 = jnp.maximum(h1 + b1_ref[...], 0.0)
    y = x + jnp.dot(h1, w2_ref[...], preferred_element_type=_f32) + b2_ref[...]
    feat_ref[...] = jnp.dot(pool_ref[...], y, preferred_element_type=_f32)
    pp = jnp.dot(pool_ref[...], pos_ref[...], preferred_element_type=_f32)
    nrm = jnp.sqrt(jnp.sum(pp * pp, axis=-1, keepdims=True))
    pos_out_ref[...] = pp / (nrm + 1e-9)


def _post(xT, msgu, pos, Wo, W1a, W1b, b1r, W2, b2r, PoolM):
    grid = (N // NB5,)
    return pl.pallas_call(
        _post_body,
        grid=grid,
        in_specs=[pl.BlockSpec((NB5, D), lambda i: (i, 0)),
                  pl.BlockSpec((NC, NB5, AW), lambda i: (0, i, 0)),
                  pl.BlockSpec((NB5, 3), lambda i: (i, 0)),
                  pl.BlockSpec((D, D), lambda i: (0, 0)),
                  pl.BlockSpec((D, 2 * D), lambda i: (0, 0)),
                  pl.BlockSpec((D, 2 * D), lambda i: (0, 0)),
                  pl.BlockSpec((1, 2 * D), lambda i: (0, 0)),
                  pl.BlockSpec((2 * D, D), lambda i: (0, 0)),
                  pl.BlockSpec((1, D), lambda i: (0, 0)),
                  pl.BlockSpec((PB, NB5), lambda i: (0, 0))],
        out_specs=[pl.BlockSpec((PB, D), lambda i: (i, 0)),
                   pl.BlockSpec((PB, 3), lambda i: (i, 0))],
        out_shape=[jax.ShapeDtypeStruct((N // 4, D), _f32),
                   jax.ShapeDtypeStruct((N // 4, 3), _f32)],
    )(xT, msgu, pos, Wo, W1a, W1b, b1r, W2, b2r, PoolM)


def _consts():
    Bsel = np.zeros((D, H), np.float32)
    for j in range(D):
        Bsel[j, j // DH] = 1.0
    Rsel = Bsel.T.copy()
    PoolM = np.kron(np.eye(PB, dtype=np.float32),
                    np.full((1, 4), 0.25, np.float32))
    return jnp.asarray(Bsel), jnp.asarray(Rsel), jnp.asarray(PoolM)


def _one_image(xT, edges, pos, Wq, Wk, Wv, Wo, W1a, W1b, b1r, W2, b2r,
               Bsel, Rsel, PoolM, zrows):
    src = edges[0]
    dst = edges[1]
    dsti_g = dst.reshape(NW, GCHUNKS, C)
    srci_g = src.reshape(NW, GCHUNKS, C)
    dsti_s = dst.reshape(NS, SCHUNKS, C)
    q, kv = _qkv(xT, Wq, Wk, Wv)
    qd, kvs = _gather(q, kv, dsti_g, srci_g)
    wma = _score(qd, kvs, Bsel, Rsel)
    msgu = _scatter(wma, dsti_s, zrows)
    feat, ppos = _post(xT, msgu, pos, Wo, W1a, W1b, b1r, W2, b2r, PoolM)
    return feat, ppos


def kernel(x0, x1, edges1, edges2, pos0, pos1, Wq, Wk, Wv, Wo, W1, b1, W2, b2):
    x0T = x0[0].T
    x1T = x1[0].T
    W1a = W1[:D]
    W1b = W1[D:]
    b1r = b1.reshape(1, 2 * D)
    b2r = b2.reshape(1, D)
    Bsel, Rsel, PoolM = _consts()
    zrows = jnp.zeros((CH, AW), _f32)
    f0, p0 = _one_image(x0T, edges1, pos0[0], Wq, Wk, Wv, Wo, W1a, W1b,
                        b1r, W2, b2r, Bsel, Rsel, PoolM, zrows)
    f1, p1 = _one_image(x1T, edges2, pos1[0], Wq, Wk, Wv, Wo, W1a, W1b,
                        b1r, W2, b2r, Bsel, Rsel, PoolM, zrows)
    return (f0[None], f1[None], p0[None], p1[None])


# speedup vs baseline: 30.6389x; 1.3959x over previous
import functools

import jax
import jax.numpy as jnp
import numpy as np
from jax import lax
from jax.experimental import pallas as pl
from jax.experimental.pallas import tpu as pltpu
from jax.experimental.pallas import tpu_sc as plsc

N = 49152
E = 589824
D = 128
H = 4
DH = 32

NC, NS = 2, 16
NW = NC * NS
C = 128
EPW = E // NW
GCHUNKS = EPW // C
EPT = E // NS
SCHUNKS = EPT // C
AW = 128
CH = C // 2
ZB = 16
NWIN = 4
NH = N // NWIN
ACC_ROWS = NH + 1024
ZPT = ACC_ROWS // NS
DPT = NH // NS

NB1 = 2048
EB = 4096
NB5 = 2048
PB = NB5 // 4

_f32 = jnp.float32


def _qkv_body(x_ref, wq_ref, wk_ref, wv_ref, q_ref, kv_ref):
    x = x_ref[...]
    q_ref[...] = jnp.dot(x, wq_ref[...], preferred_element_type=_f32)
    k = jnp.dot(x, wk_ref[...], preferred_element_type=_f32)
    v = jnp.dot(x, wv_ref[...], preferred_element_type=_f32)
    kv_ref[...] = jnp.concatenate([k, v], axis=-1)


def _qkv(xT, Wq, Wk, Wv):
    grid = (N // NB1,)
    wspec = pl.BlockSpec((D, D), lambda i: (0, 0))
    return pl.pallas_call(
        _qkv_body,
        grid=grid,
        in_specs=[pl.BlockSpec((NB1, D), lambda i: (i, 0)), wspec, wspec, wspec],
        out_specs=[pl.BlockSpec((NB1, D), lambda i: (i, 0)),
                   pl.BlockSpec((NB1, 2 * D), lambda i: (i, 0))],
        out_shape=[jax.ShapeDtypeStruct((N, D), _f32),
                   jax.ShapeDtypeStruct((N, 2 * D), _f32)],
    )(xT, Wq, Wk, Wv)


def _gather_body(q_hbm, kv_hbm, dsti_hbm, srci_hbm, qd_hbm, kvs_hbm,
                 didx, sidx, qbuf, kvbuf, sem_q, sem_kv):
    wid = lax.axis_index("s") * NC + lax.axis_index("c")
    pltpu.sync_copy(dsti_hbm.at[wid], didx)
    pltpu.sync_copy(srci_hbm.at[wid], sidx)

    def step(j, carry):
        base = wid * EPW + j * C
        cq = pltpu.async_copy(q_hbm.at[didx.at[j]], qbuf, sem_q)
        ckv = pltpu.async_copy(kv_hbm.at[sidx.at[j]], kvbuf, sem_kv)
        cq.wait()
        pltpu.sync_copy(qbuf, qd_hbm.at[pl.ds(base, C)])
        ckv.wait()
        pltpu.sync_copy(kvbuf, kvs_hbm.at[pl.ds(base, C)])
        return carry

    lax.fori_loop(0, GCHUNKS, step, 0)


def _gather(q, kv, dsti, srci):
    mesh = plsc.VectorSubcoreMesh(core_axis_name="c", subcore_axis_name="s",
                                  num_cores=NC, num_subcores=NS)
    f = pl.kernel(
        _gather_body,
        out_type=[jax.ShapeDtypeStruct((E, D), _f32),
                  jax.ShapeDtypeStruct((E, 2 * D), _f32)],
        mesh=mesh,
        scratch_types=[
            pltpu.VMEM((GCHUNKS, C), jnp.int32),
            pltpu.VMEM((GCHUNKS, C), jnp.int32),
            pltpu.VMEM((C, D), _f32),
            pltpu.VMEM((C, 2 * D), _f32),
            pltpu.SemaphoreType.DMA,
            pltpu.SemaphoreType.DMA,
        ],
    )
    return f(q, kv, dsti, srci)


def _score_body(qd_ref, kvs_ref, b_ref, r_ref, out_ref):
    qd = qd_ref[...]
    kvs = kvs_ref[...]
    k = kvs[:, :D]
    v = kvs[:, D:]
    prod = qd * k
    s = jnp.dot(prod, b_ref[...], preferred_element_type=_f32) * (1.0 / np.sqrt(DH))
    e = jnp.exp(s)
    eexp = jnp.dot(e, r_ref[...], preferred_element_type=_f32)
    wm = v * eexp
    zz = jnp.zeros((EB, DH - 1), _f32)
    for c in range(NC):
        row = jnp.concatenate(
            [wm[:, DH * 2 * c:DH * (2 * c + 1)], e[:, 2 * c:2 * c + 1], zz,
             wm[:, DH * (2 * c + 1):DH * (2 * c + 2)],
             e[:, 2 * c + 1:2 * c + 2], zz], axis=-1)
        out_ref[c] = row


def _score(qd, kvs, Bsel, Rsel):
    grid = (E // EB,)
    return pl.pallas_call(
        _score_body,
        grid=grid,
        in_specs=[pl.BlockSpec((EB, D), lambda i: (i, 0)),
                  pl.BlockSpec((EB, 2 * D), lambda i: (i, 0)),
                  pl.BlockSpec((D, H), lambda i: (0, 0)),
                  pl.BlockSpec((H, D), lambda i: (0, 0))],
        out_specs=pl.BlockSpec((NC, EB, AW), lambda i: (0, i, 0)),
        out_shape=jax.ShapeDtypeStruct((NC, E, AW), _f32),
    )(qd, kvs, Bsel, Rsel)


def _scatter_body(wma_hbm, dsti_hbm, z_hbm, out_hbm,
                  idxc, idxta, idxtb, bufa, bufb, zv, acc, sema, semb):
    c = lax.axis_index("c")
    sid = lax.axis_index("s")
    ebase = sid * EPT
    pltpu.sync_copy(z_hbm, zv)
    for w in range(NWIN):
        base = w * NH

        def zero_step(i, carry):
            pltpu.sync_copy(zv, acc.at[pl.ds(sid * ZPT + i * ZB, ZB)])
            return carry

        lax.fori_loop(0, ZPT // ZB, zero_step, 0)
        plsc.subcore_barrier()

        pltpu.async_copy(wma_hbm.at[c, pl.ds(ebase, CH)], bufa, sema)

        def step(j, carry):
            e0 = ebase + j * C
            pltpu.async_copy(wma_hbm.at[c, pl.ds(e0 + CH, CH)], bufb, semb)
            pltpu.sync_copy(dsti_hbm.at[sid, j], idxc)
            for u in range(CH // 16):
                a = idxc[pl.ds(u * 16, 16)]
                t = a - base
                ok = (t >= 0) & (t < NH)
                idxta[pl.ds(u * 16, 16)] = jnp.where(ok, t, NH)
                a2 = idxc[pl.ds(CH + u * 16, 16)]
                t2 = a2 - base
                ok2 = (t2 >= 0) & (t2 < NH)
                idxtb[pl.ds(u * 16, 16)] = jnp.where(ok2, t2, NH)
            pltpu.make_async_copy(wma_hbm.at[c, pl.ds(e0, CH)], bufa,
                                  sema).wait()
            pltpu.sync_copy(bufa, acc.at[idxta], add=True)
            en = lax.min(e0 + C, EPT * NS - CH)
            pltpu.async_copy(wma_hbm.at[c, pl.ds(en, CH)], bufa, sema)
            pltpu.make_async_copy(wma_hbm.at[c, pl.ds(e0 + CH, CH)], bufb,
                                  semb).wait()
            pltpu.sync_copy(bufb, acc.at[idxtb], add=True)
            return carry

        lax.fori_loop(0, SCHUNKS, step, 0)
        pltpu.make_async_copy(wma_hbm.at[c, pl.ds(0, CH)], bufa, sema).wait()
        plsc.subcore_barrier()
        pltpu.sync_copy(acc.at[pl.ds(sid * DPT, DPT)],
                        out_hbm.at[c, pl.ds(base + sid * DPT, DPT)])
        plsc.subcore_barrier()


def _scatter(wma, dsti, zrows):
    mesh = plsc.VectorSubcoreMesh(core_axis_name="c", subcore_axis_name="s",
                                  num_cores=NC, num_subcores=NS)
    f = pl.kernel(
        _scatter_body,
        out_type=jax.ShapeDtypeStruct((NC, N, AW), _f32),
        mesh=mesh,
        scratch_types=[
            pltpu.VMEM((C,), jnp.int32),
            pltpu.VMEM((CH,), jnp.int32),
            pltpu.VMEM((CH,), jnp.int32),
            pltpu.VMEM((CH, AW), _f32),
            pltpu.VMEM((CH, AW), _f32),
            pltpu.VMEM((ZB, AW), _f32),
            pltpu.VMEM_SHARED((ACC_ROWS, AW), _f32),
            pltpu.SemaphoreType.DMA,
            pltpu.SemaphoreType.DMA,
        ],
    )
    return f(wma, dsti, zrows)


def _post_body(x_ref, msgu_ref, pos_ref, wo_ref, w1a_ref, w1b_ref, b1_ref,
               w2_ref, b2_ref, pool_ref, feat_ref, pos_out_ref):
    x = x_ref[...]
    mu = msgu_ref[...]
    pieces = []
    for c in range(NC):
        bh = mu[c]
        for p in range(2):
            num = bh[:, 64 * p:64 * p + DH]
            den = bh[:, 64 * p + DH:64 * p + DH + 1] + 1e-9
            pieces.append(num / den)
    msg = jnp.concatenate(pieces, axis=-1)
    mo = jnp.dot(msg, wo_ref[...], preferred_element_type=_f32)
    h1 = jnp.dot(x, w1a_ref[...], preferred_element_type=_f32)
    h1 = h1 + jnp.dot(mo, w1b_ref[...], preferred_element_type=_f32)
    h1 = jnp.maximum(h1 + b1_ref[...], 0.0)
    y = x + jnp.dot(h1, w2_ref[...], preferred_element_type=_f32) + b2_ref[...]
    feat_ref[...] = jnp.dot(pool_ref[...], y, preferred_element_type=_f32)
    pp = jnp.dot(pool_ref[...], pos_ref[...], preferred_element_type=_f32)
    nrm = jnp.sqrt(jnp.sum(pp * pp, axis=-1, keepdims=True))
    pos_out_ref[...] = pp / (nrm + 1e-9)


def _post(xT, msgu, pos, Wo, W1a, W1b, b1r, W2, b2r, PoolM):
    grid = (N // NB5,)
    return pl.pallas_call(
        _post_body,
        grid=grid,
        in_specs=[pl.BlockSpec((NB5, D), lambda i: (i, 0)),
                  pl.BlockSpec((NC, NB5, AW), lambda i: (0, i, 0)),
                  pl.BlockSpec((NB5, 3), lambda i: (i, 0)),
                  pl.BlockSpec((D, D), lambda i: (0, 0)),
                  pl.BlockSpec((D, 2 * D), lambda i: (0, 0)),
                  pl.BlockSpec((D, 2 * D), lambda i: (0, 0)),
                  pl.BlockSpec((1, 2 * D), lambda i: (0, 0)),
                  pl.BlockSpec((2 * D, D), lambda i: (0, 0)),
                  pl.BlockSpec((1, D), lambda i: (0, 0)),
                  pl.BlockSpec((PB, NB5), lambda i: (0, 0))],
        out_specs=[pl.BlockSpec((PB, D), lambda i: (i, 0)),
                   pl.BlockSpec((PB, 3), lambda i: (i, 0))],
        out_shape=[jax.ShapeDtypeStruct((N // 4, D), _f32),
                   jax.ShapeDtypeStruct((N // 4, 3), _f32)],
    )(xT, msgu, pos, Wo, W1a, W1b, b1r, W2, b2r, PoolM)


def _consts():
    Bsel = np.zeros((D, H), np.float32)
    for j in range(D):
        Bsel[j, j // DH] = 1.0
    Rsel = Bsel.T.copy()
    PoolM = np.kron(np.eye(PB, dtype=np.float32),
                    np.full((1, 4), 0.25, np.float32))
    return jnp.asarray(Bsel), jnp.asarray(Rsel), jnp.asarray(PoolM)


def _one_image(xT, edges, pos, Wq, Wk, Wv, Wo, W1a, W1b, b1r, W2, b2r,
               Bsel, Rsel, PoolM, zrows):
    src = edges[0]
    dst = edges[1]
    dsti_g = dst.reshape(NW, GCHUNKS, C)
    srci_g = src.reshape(NW, GCHUNKS, C)
    dsti_s = dst.reshape(NS, SCHUNKS, C)
    q, kv = _qkv(xT, Wq, Wk, Wv)
    qd, kvs = _gather(q, kv, dsti_g, srci_g)
    wma = _score(qd, kvs, Bsel, Rsel)
    msgu = _scatter(wma, dsti_s, zrows)
    feat, ppos = _post(xT, msgu, pos, Wo, W1a, W1b, b1r, W2, b2r, PoolM)
    return feat, ppos


def kernel(x0, x1, edges1, edges2, pos0, pos1, Wq, Wk, Wv, Wo, W1, b1, W2, b2):
    x0T = x0[0].T
    x1T = x1[0].T
    W1a = W1[:D]
    W1b = W1[D:]
    b1r = b1.reshape(1, 2 * D)
    b2r = b2.reshape(1, D)
    Bsel, Rsel, PoolM = _consts()
    zrows = jnp.zeros((ZB, AW), _f32)
    f0, p0 = _one_image(x0T, edges1, pos0[0], Wq, Wk, Wv, Wo, W1a, W1b,
                        b1r, W2, b2r, Bsel, Rsel, PoolM, zrows)
    f1, p1 = _one_image(x1T, edges2, pos1[0], Wq, Wk, Wv, Wo, W1a, W1b,
                        b1r, W2, b2r, Bsel, Rsel, PoolM, zrows)
    return (f0[None], f1[None], p0[None], p1[None])
